# baseline (speedup 1.0000x reference)
import functools

import jax
import jax.numpy as jnp
from jax import lax
from jax.experimental import pallas as pl
from jax.experimental.pallas import tpu as pltpu
from jax.experimental.pallas import tpu_sc as plsc

_B, _T, _C = 16, 1024, 999
_N = _B * _T
_ROWS = 1024

_NC, _NS, _L = 2, 16, 16
_NW = _NC * _NS
_PW = _N // _NW
_CHUNKS = _PW // _L


def _tc_body(x_ref, o_ref):
    x = x_ref[...]

    @pl.when(pl.program_id(0) == 0)
    def _():
        o_ref[0, 0] = 0.0

    o_ref[0, 0] += jnp.sum(x * x)


_sc_mesh = plsc.VectorSubcoreMesh(core_axis_name="c", subcore_axis_name="s")


@functools.partial(
    pl.kernel,
    mesh=_sc_mesh,
    out_type=jax.ShapeDtypeStruct((_NW, _L), jnp.float32),
    scratch_types=[
        pltpu.VMEM((_PW,), jnp.int32),
        pltpu.VMEM((_PW,), jnp.int32),
        pltpu.VMEM((_PW,), jnp.float32),
        pltpu.VMEM((_L,), jnp.float32),
        pltpu.SemaphoreType.DMA,
    ],
)
def _sc_cross(t_hbm, x_hbm, out_hbm, t_v, idx_v, val_v, acc_v, sem):
    wid = lax.axis_index("s") * _NC + lax.axis_index("c")
    base = wid * _PW
    pltpu.sync_copy(t_hbm.at[pl.ds(base, _PW)], t_v)

    def mk_idx(j, carry):
        tt = t_v[pl.ds(j * _L, _L)]
        row = base + j * _L + lax.iota(jnp.int32, _L)
        idx_v[pl.ds(j * _L, _L)] = row * _C + jnp.maximum(tt - 1, 0)
        return carry

    lax.fori_loop(0, _CHUNKS, mk_idx, 0)
    pltpu.async_copy(x_hbm.at[idx_v], val_v, sem).wait()

    def acc_step(j, acc):
        tt = t_v[pl.ds(j * _L, _L)]
        v = val_v[pl.ds(j * _L, _L)]
        return acc + jnp.where(tt >= 1, v, 0.0)

    acc_v[...] = lax.fori_loop(0, _CHUNKS, acc_step,
                               jnp.zeros((_L,), jnp.float32))
    pltpu.sync_copy(acc_v, out_hbm.at[wid])


def kernel(rel_ress, targets, mask):
    del mask
    x2 = rel_ress.reshape(_N, _C)
    cross_parts = _sc_cross(targets.astype(jnp.int32).reshape(_N),
                            rel_ress.reshape(_N * _C))
    sumsq = pl.pallas_call(
        _tc_body,
        grid=(_N // _ROWS,),
        in_specs=[pl.BlockSpec((_ROWS, _C), lambda i: (i, 0))],
        out_specs=pl.BlockSpec(memory_space=pltpu.SMEM),
        out_shape=jax.ShapeDtypeStruct((1, 1), jnp.float32),
    )(x2)
    cross = jnp.sum(cross_parts)
    return (sumsq[0, 0] - 2.0 * cross + jnp.float32(_N)) / jnp.float32(_N * (_C + 1))


# speedup vs baseline: 1.6800x; 1.6800x over previous
import jax
import jax.numpy as jnp
from jax import lax
from jax.experimental import pallas as pl
from jax.experimental.pallas import tpu as pltpu

_B, _T, _C = 16, 1024, 999
_N = _B * _T
_OPS = 4
_ROWS = 512
_STEPS = _N // (_OPS * _ROWS)


def _body(*refs):
    x_refs = refs[:_OPS]
    t_refs = refs[_OPS:2 * _OPS]
    o_ref = refs[2 * _OPS]
    col = lax.broadcasted_iota(jnp.int32, (_ROWS, _C), 1)
    part = jnp.float32(0.0)
    for x_ref, t_ref in zip(x_refs, t_refs):
        x = x_ref[...]
        t = t_ref[...]
        hit = col == (t - 1)
        part += jnp.sum(x * x) - 2.0 * jnp.sum(jnp.where(hit, x, 0.0))

    @pl.when(pl.program_id(0) == 0)
    def _():
        o_ref[0, 0] = 0.0

    o_ref[0, 0] += part


def kernel(rel_ress, targets, mask):
    del mask
    x = rel_ress.reshape(_N, _C)
    t_col = targets.astype(jnp.int32).reshape(_N, 1)
    x_specs = [
        pl.BlockSpec((_ROWS, _C), lambda i, k=k: (i + k * _STEPS, 0))
        for k in range(_OPS)
    ]
    t_specs = [
        pl.BlockSpec((_ROWS, 1), lambda i, k=k: (i + k * _STEPS, 0))
        for k in range(_OPS)
    ]
    out = pl.pallas_call(
        _body,
        grid=(_STEPS,),
        in_specs=x_specs + t_specs,
        out_specs=pl.BlockSpec(memory_space=pltpu.SMEM),
        out_shape=jax.ShapeDtypeStruct((1, 1), jnp.float32),
    )(*([x] * _OPS + [t_col] * _OPS))
    return (out[0, 0] + jnp.float32(_N)) / jnp.float32(_N * (_C + 1))
